# baseline (speedup 1.0000x reference)
import functools

import jax
import jax.numpy as jnp
from jax import lax
from jax.experimental import pallas as pl
from jax.experimental.pallas import tpu as pltpu
from jax.experimental.pallas import tpu_sc as plsc

B = 16384
D = 16
H = 50
MROWS = 400
MBAGS = MROWS // H


def _sc_info():
    try:
        info = plsc.get_sparse_core_info()
        return info.num_cores, info.num_subcores
    except Exception:
        return 2, 16


def _body(uidx, hidx, pvals, wvec, wu, wi, wh,
          out_u, out_i, out_h, out_p,
          uidx_v, ustage, hidx_v, hbufa, hbufb, hstage, pv, wv,
          sem_u, sema, semb, semi, nc, bpt):
    wid = lax.axis_index("s") * nc + lax.axis_index("c")
    base = wid * bpt
    rows = bpt * H
    n_macro = rows // MROWS
    half = n_macro // 2

    pltpu.sync_copy(uidx.at[wid], uidx_v)
    pltpu.sync_copy(hidx.at[wid, 0], hidx_v)
    pltpu.sync_copy(pvals.at[wid], pv)
    pltpu.sync_copy(wvec, wv)

    def fire_macro(m, buf, sem):
        p0 = (m % half) * MROWS

        def fbody(j, carry):
            p = p0 + j * 16
            idx16 = hidx_v[p // 128, pl.ds(p % 128, 16)]
            for k in range(16):
                pltpu.make_async_copy(
                    wh.at[idx16[k]], buf.at[j * 16 + k], sem
                ).start()
            return carry

        lax.fori_loop(0, MROWS // 16, fbody, 0)

    def drain_macro(buf, sem):
        pltpu.make_async_copy(wh.at[pl.ds(0, MROWS)], buf, sem).wait()

    fire_macro(0, hbufa, sema)
    fire_macro(1, hbufb, semb)

    w = wv[0, pl.ds(0, 16)]

    def pbody(i, carry):
        p = i * 16
        p16 = pv[p // 128, pl.ds(p % 128, 16)]
        for k in range(16):
            ustage[k] = w * p16[k]
        pltpu.sync_copy(ustage, out_p.at[pl.ds(base + p, 16)])
        return carry

    lax.fori_loop(0, bpt // 16, pbody, 0)

    def lookup(t, table, out):
        def cbody(c, carry):
            idx16 = uidx_v[t, pl.ds(c * 16, 16)]
            for k in range(16):
                pltpu.make_async_copy(
                    table.at[idx16[k]], ustage.at[k], sem_u
                ).start()
            for k in range(16):
                pltpu.make_async_copy(
                    table.at[idx16[k]], ustage.at[k], sem_u
                ).wait()
            pltpu.sync_copy(ustage, out.at[pl.ds(base + c * 16, 16)])
            return carry

        lax.fori_loop(0, bpt // 16, cbody, 0)

    lookup(0, wu, out_u)
    lookup(1, wi, out_i)

    inv_h = 1.0 / H

    def process_macro(m, buf):
        def cbody(k, carry):
            gb = k * H
            acc0 = buf[gb]
            acc1 = buf[gb + 1]
            for h in range(2, H, 2):
                acc0 = acc0 + buf[gb + h]
                acc1 = acc1 + buf[gb + h + 1]
            hstage[k] = (acc0 + acc1) * inv_h
            return carry

        lax.fori_loop(0, MBAGS, cbody, 0)
        pltpu.sync_copy(hstage, out_h.at[pl.ds(base + m * MBAGS, MBAGS)])

    bufs = (hbufa, hbufb)
    sems = (sema, semb)
    stage_b = pltpu.make_async_copy(hidx.at[wid, 1], hidx_v, semi)

    def hbody(i, carry):
        for b in range(2):
            m = 2 * i + b
            drain_macro(bufs[b], sems[b])
            process_macro(m, bufs[b])
            nxt = m + 2

            @pl.when(nxt == half)
            def _():
                stage_b.wait()

            @pl.when(nxt < n_macro)
            def _():
                fire_macro(nxt, bufs[b], sems[b])

            @pl.when(nxt == half - 1)
            def _():
                stage_b.start()

        return carry

    lax.fori_loop(0, n_macro // 2, hbody, 0)


def kernel(user_id, item_id, item_hist, price, W_user, W_item, W_hist, W_price):
    nc, ns = _sc_info()
    nw = nc * ns
    bpt = B // nw
    rows = bpt * H

    uidx = jnp.stack(
        [user_id.astype(jnp.int32).reshape(nw, bpt),
         item_id.astype(jnp.int32).reshape(nw, bpt)], axis=1)
    hidx = item_hist.astype(jnp.int32).reshape(nw, 2, rows // 256, 128)
    pvals = price.reshape(nw, bpt // 128, 128)
    wvec = jnp.zeros((1, 128), jnp.float32).at[0, :D].set(W_price.reshape(D))

    mesh = plsc.VectorSubcoreMesh(core_axis_name="c", subcore_axis_name="s")
    f32 = jnp.float32
    sds = jax.ShapeDtypeStruct
    grid_kernel = pl.kernel(
        functools.partial(_body, nc=nc, bpt=bpt),
        mesh=mesh,
        out_type=(
            sds((B, D), f32),
            sds((B, D), f32),
            sds((B, D), f32),
            sds((B, D), f32),
        ),
        scratch_types=[
            pltpu.VMEM((2, bpt), jnp.int32),
            pltpu.VMEM((16, D), f32),
            pltpu.VMEM((rows // 256, 128), jnp.int32),
            pltpu.VMEM((MROWS, D), f32),
            pltpu.VMEM((MROWS, D), f32),
            pltpu.VMEM((MBAGS, D), f32),
            pltpu.VMEM((bpt // 128, 128), f32),
            pltpu.VMEM((1, 128), f32),
            pltpu.SemaphoreType.DMA,
            pltpu.SemaphoreType.DMA,
            pltpu.SemaphoreType.DMA,
            pltpu.SemaphoreType.DMA,
        ],
    )
    return grid_kernel(uidx, hidx, pvals, wvec, W_user, W_item, W_hist)


# speedup vs baseline: 1.0788x; 1.0788x over previous
import functools

import jax
import jax.numpy as jnp
from jax import lax
from jax.experimental import pallas as pl
from jax.experimental.pallas import tpu as pltpu
from jax.experimental.pallas import tpu_sc as plsc

B = 16384
D = 16
H = 50
MACRO_ROWS = 1600
MACRO_BAGS = MACRO_ROWS // H


def _sc_info():
    try:
        info = plsc.get_sparse_core_info()
        return info.num_cores, info.num_subcores
    except Exception:
        return 2, 16


def _hist_body(hidx, pvals, wvec, wh,
               out_h, out_p,
               hidx_v, hbufa, hbufb, outv, pbuf, pv, wv,
               sema, semb, nc, bpt):
    wid = lax.axis_index("s") * nc + lax.axis_index("c")
    base = wid * bpt
    rows = bpt * H
    n_macro = rows // MACRO_ROWS

    pltpu.sync_copy(hidx.at[wid], hidx_v)
    pltpu.sync_copy(pvals.at[wid], pv)
    pltpu.sync_copy(wvec, wv)

    def fire_macro(m, buf, sem):
        def fbody(j, carry):
            idxv = hidx_v[pl.ds(m * MACRO_ROWS + j * 16, 16)]
            pltpu.make_async_copy(
                wh.at[idxv], buf.at[pl.ds(j * 16, 16)], sem
            ).start()
            return carry

        lax.fori_loop(0, MACRO_ROWS // 16, fbody, 0)

    def drain_macro(m, buf, sem):
        pltpu.make_async_copy(
            wh.at[hidx_v.at[pl.ds(m * MACRO_ROWS, MACRO_ROWS)]], buf, sem
        ).wait()

    fire_macro(0, hbufa, sema)
    fire_macro(1, hbufb, semb)

    w = wv[...]

    def pbody(i, carry):
        p16 = pv[pl.ds(i * 16, 16)]
        for k in range(16):
            pbuf[i * 16 + k] = w * p16[k]
        return carry

    lax.fori_loop(0, bpt // 16, pbody, 0)
    pltpu.sync_copy(pbuf, out_p.at[pl.ds(base, bpt)])

    inv_h = 1.0 / H

    def process_macro(m, buf):
        def cbody(k, carry):
            gb = k * H
            acc0 = buf[gb]
            acc1 = buf[gb + 1]
            for h in range(2, H, 2):
                acc0 = acc0 + buf[gb + h]
                acc1 = acc1 + buf[gb + h + 1]
            outv[m * MACRO_BAGS + k] = (acc0 + acc1) * inv_h
            return carry

        lax.fori_loop(0, MACRO_BAGS, cbody, 0)

    bufs = (hbufa, hbufb)
    sems = (sema, semb)

    def hbody(i, carry):
        for b in range(2):
            m = 2 * i + b
            drain_macro(m, bufs[b], sems[b])
            process_macro(m, bufs[b])
            nxt = m + 2

            @pl.when(nxt < n_macro)
            def _():
                fire_macro(nxt, bufs[b], sems[b])

        return carry

    lax.fori_loop(0, n_macro // 2, hbody, 0)
    pltpu.sync_copy(outv, out_h.at[pl.ds(base, bpt)])


def _ui_body(uidx, wu, wi,
             out_u, out_i,
             uidx_v, ubuf, sem_u, nc, bpt):
    wid = lax.axis_index("s") * nc + lax.axis_index("c")
    base = wid * bpt

    pltpu.sync_copy(uidx.at[wid], uidx_v)

    def lookup(t, table, out):
        def cbody(c, carry):
            idx16 = uidx_v[t, pl.ds(c * 16, 16)]
            for k in range(16):
                pltpu.make_async_copy(
                    table.at[idx16[k]], ubuf.at[c * 16 + k], sem_u
                ).start()
            for k in range(16):
                pltpu.make_async_copy(
                    table.at[idx16[k]], ubuf.at[c * 16 + k], sem_u
                ).wait()
            return carry

        lax.fori_loop(0, bpt // 16, cbody, 0)
        pltpu.sync_copy(ubuf, out.at[pl.ds(base, bpt)])

    lookup(0, wu, out_u)
    lookup(1, wi, out_i)


def kernel(user_id, item_id, item_hist, price, W_user, W_item, W_hist, W_price):
    nc, ns = _sc_info()
    nw = nc * ns
    bpt = B // nw
    rows = bpt * H

    uidx = jnp.stack(
        [user_id.astype(jnp.int32).reshape(nw, bpt),
         item_id.astype(jnp.int32).reshape(nw, bpt)], axis=1)
    hidx = item_hist.astype(jnp.int32).reshape(nw, rows)
    pvals = price.reshape(nw, bpt)
    wvec = W_price.reshape(D)

    mesh = plsc.VectorSubcoreMesh(core_axis_name="c", subcore_axis_name="s")
    f32 = jnp.float32
    sds = jax.ShapeDtypeStruct

    hist_kernel = pl.kernel(
        functools.partial(_hist_body, nc=nc, bpt=bpt),
        mesh=mesh,
        compiler_params=pltpu.CompilerParams(use_tc_tiling_on_sc=False),
        out_type=(
            sds((B, D), f32),
            sds((B, D), f32),
        ),
        scratch_types=[
            pltpu.VMEM((rows,), jnp.int32),
            pltpu.VMEM((MACRO_ROWS, D), f32),
            pltpu.VMEM((MACRO_ROWS, D), f32),
            pltpu.VMEM((bpt, D), f32),
            pltpu.VMEM((bpt, D), f32),
            pltpu.VMEM((bpt,), f32),
            pltpu.VMEM((D,), f32),
            pltpu.SemaphoreType.DMA,
            pltpu.SemaphoreType.DMA,
        ],
    )
    ui_kernel = pl.kernel(
        functools.partial(_ui_body, nc=nc, bpt=bpt),
        mesh=mesh,
        out_type=(
            sds((B, D), f32),
            sds((B, D), f32),
        ),
        scratch_types=[
            pltpu.VMEM((2, bpt), jnp.int32),
            pltpu.VMEM((bpt, D), f32),
            pltpu.SemaphoreType.DMA,
        ],
    )

    emb_hist, emb_price = hist_kernel(hidx, pvals, wvec, W_hist)
    emb_user, emb_item = ui_kernel(uidx, W_user, W_item)
    return emb_user, emb_item, emb_hist, emb_price


# speedup vs baseline: 1.0819x; 1.0028x over previous
import functools

import jax
import jax.numpy as jnp
from jax import lax
from jax.experimental import pallas as pl
from jax.experimental.pallas import tpu as pltpu
from jax.experimental.pallas import tpu_sc as plsc

B = 16384
D = 16
H = 50
MACRO_ROWS = 1600
MACRO_BAGS = MACRO_ROWS // H


def _sc_info():
    try:
        info = plsc.get_sparse_core_info()
        return info.num_cores, info.num_subcores
    except Exception:
        return 2, 16


def _hist_body(hidx, pvals, wvec, wh,
               out_h, out_p,
               hidx_v, hbufa, hbufb, outv, pbuf, pv, wv,
               sema, semb, nc, bpt):
    wid = lax.axis_index("s") * nc + lax.axis_index("c")
    base = wid * bpt
    rows = bpt * H
    n_macro = rows // MACRO_ROWS

    pltpu.sync_copy(hidx.at[wid], hidx_v)
    pltpu.sync_copy(pvals.at[wid], pv)
    pltpu.sync_copy(wvec, wv)

    def fire_macro(m, buf, sem):
        def fbody(j, carry):
            idxv = hidx_v[pl.ds(m * MACRO_ROWS + j * 16, 16)]
            pltpu.make_async_copy(
                wh.at[idxv], buf.at[pl.ds(j * 16, 16)], sem
            ).start()
            return carry

        lax.fori_loop(0, MACRO_ROWS // 16, fbody, 0)

    def drain_macro(m, buf, sem):
        pltpu.make_async_copy(
            wh.at[hidx_v.at[pl.ds(m * MACRO_ROWS, MACRO_ROWS)]], buf, sem
        ).wait()

    fire_macro(0, hbufa, sema)
    fire_macro(1, hbufb, semb)

    w = wv[...]

    def pbody(i, carry):
        p16 = pv[pl.ds(i * 16, 16)]
        for k in range(16):
            pbuf[i * 16 + k] = w * p16[k]
        return carry

    lax.fori_loop(0, bpt // 16, pbody, 0)
    pltpu.sync_copy(pbuf, out_p.at[pl.ds(base, bpt)])

    inv_h = 1.0 / H

    def process_macro(m, buf):
        def cbody(k, carry):
            gb = k * H
            acc0 = buf[gb]
            acc1 = buf[gb + 1]
            for h in range(2, H, 2):
                acc0 = acc0 + buf[gb + h]
                acc1 = acc1 + buf[gb + h + 1]
            outv[m * MACRO_BAGS + k] = (acc0 + acc1) * inv_h
            return carry

        lax.fori_loop(0, MACRO_BAGS, cbody, 0)

    bufs = (hbufa, hbufb)
    sems = (sema, semb)

    def hbody(i, carry):
        for b in range(2):
            m = 2 * i + b
            drain_macro(m, bufs[b], sems[b])
            process_macro(m, bufs[b])
            nxt = m + 2

            @pl.when(nxt < n_macro)
            def _():
                fire_macro(nxt, bufs[b], sems[b])

        return carry

    lax.fori_loop(0, n_macro // 2, hbody, 0)
    pltpu.sync_copy(outv, out_h.at[pl.ds(base, bpt)])


def _ui_body(uidx, wu, wi,
             out_u, out_i,
             uidx_v, ubuf, sem_u, nc, bpt):
    wid = lax.axis_index("s") * nc + lax.axis_index("c")
    base = wid * bpt

    pltpu.sync_copy(uidx.at[wid], uidx_v)

    def fire(t, table, buf, sem):
        def cbody(c, carry):
            idx16 = uidx_v[t, pl.ds(c * 16, 16)]
            for k in range(16):
                pltpu.make_async_copy(
                    table.at[idx16[k]], buf.at[c * 16 + k], sem
                ).start()
            return carry

        lax.fori_loop(0, bpt // 16, cbody, 0)

    fire(0, wu, ubuf, sem_u)
    pltpu.make_async_copy(wu.at[pl.ds(0, bpt)], ubuf, sem_u).wait()
    pltpu.sync_copy(ubuf, out_u.at[pl.ds(base, bpt)])
    fire(1, wi, ubuf, sem_u)
    pltpu.make_async_copy(wi.at[pl.ds(0, bpt)], ubuf, sem_u).wait()
    pltpu.sync_copy(ubuf, out_i.at[pl.ds(base, bpt)])


def kernel(user_id, item_id, item_hist, price, W_user, W_item, W_hist, W_price):
    nc, ns = _sc_info()
    nw = nc * ns
    bpt = B // nw
    rows = bpt * H

    uidx = jnp.stack(
        [user_id.astype(jnp.int32).reshape(nw, bpt),
         item_id.astype(jnp.int32).reshape(nw, bpt)], axis=1)
    hidx = item_hist.astype(jnp.int32).reshape(nw, rows)
    pvals = price.reshape(nw, bpt)
    wvec = W_price.reshape(D)

    mesh = plsc.VectorSubcoreMesh(core_axis_name="c", subcore_axis_name="s")
    f32 = jnp.float32
    sds = jax.ShapeDtypeStruct

    hist_kernel = pl.kernel(
        functools.partial(_hist_body, nc=nc, bpt=bpt),
        mesh=mesh,
        compiler_params=pltpu.CompilerParams(use_tc_tiling_on_sc=False),
        out_type=(
            sds((B, D), f32),
            sds((B, D), f32),
        ),
        scratch_types=[
            pltpu.VMEM((rows,), jnp.int32),
            pltpu.VMEM((MACRO_ROWS, D), f32),
            pltpu.VMEM((MACRO_ROWS, D), f32),
            pltpu.VMEM((bpt, D), f32),
            pltpu.VMEM((bpt, D), f32),
            pltpu.VMEM((bpt,), f32),
            pltpu.VMEM((D,), f32),
            pltpu.SemaphoreType.DMA,
            pltpu.SemaphoreType.DMA,
        ],
    )
    ui_kernel = pl.kernel(
        functools.partial(_ui_body, nc=nc, bpt=bpt),
        mesh=mesh,
        out_type=(
            sds((B, D), f32),
            sds((B, D), f32),
        ),
        scratch_types=[
            pltpu.VMEM((2, bpt), jnp.int32),
            pltpu.VMEM((bpt, D), f32),
            pltpu.SemaphoreType.DMA,
        ],
    )

    emb_hist, emb_price = hist_kernel(hidx, pvals, wvec, W_hist)
    emb_user, emb_item = ui_kernel(uidx, W_user, W_item)
    return emb_user, emb_item, emb_hist, emb_price
